# baseline (speedup 1.0000x reference)
import functools
import math

import numpy as np
import jax
import jax.numpy as jnp
from jax import lax
from jax.experimental import pallas as pl
from jax.experimental.pallas import tpu as pltpu
from jax.experimental.pallas import tpu_sc as plsc

N = 10000
E = 320000
IN_DIM = 128
H = 8
D = 16
HD = H * D

NC = 2
NS = 16
NW = NC * NS
EPW = E // NW
CH = 40
NCHUNK = EPW // CH
N2 = 10240
RPW = N2 // NS
NP2 = N2 // 8
PPW = NP2 // NS

_ci = np.arange(HD)
_PERM = (((_ci % 16) % 8) * D + 2 * (_ci // 16) + (_ci % 16 >= 8)).astype(np.int32)

BN = 1000
BE = 2000
BF = 1000


def _proj_body(x_ref, w_ref, b_ref, out_ref):
    xb = x_ref[...]
    for j in range(5):
        acc = lax.dot_general(xb, w_ref[j], (((1,), (1,)), ((), ())),
                              preferred_element_type=jnp.float32)
        out_ref[j] = acc + b_ref[j][None, :]


def _projections(x, wstack, bstack):
    return pl.pallas_call(
        _proj_body,
        grid=(N // BN,),
        in_specs=[
            pl.BlockSpec((BN, IN_DIM), lambda i: (i, 0)),
            pl.BlockSpec((5, HD, IN_DIM), lambda i: (0, 0, 0)),
            pl.BlockSpec((5, HD), lambda i: (0, 0)),
        ],
        out_specs=pl.BlockSpec((5, BN, HD), lambda i: (0, i, 0)),
        out_shape=jax.ShapeDtypeStruct((5, N, HD), jnp.float32),
    )(x, wstack, bstack)


def _edge_body(attr_ref, src_ref, dst_ref, we_ref, b2_ref,
               e_ref, ik_ref, iq_ref, iv_ref, ip_ref):
    attr = attr_ref[...]
    src = src_ref[0, 0]
    dst = dst_ref[0, 0]
    p = lax.dot_general(attr, we_ref[...], (((1,), (1,)), ((), ())),
                        preferred_element_type=jnp.float32)
    p = p + b2_ref[0][None, :]
    mask = jnp.max(jnp.abs(attr), axis=1) == 0.0
    e_ref[...] = jnp.where(mask[:, None], b2_ref[1][None, :], p) * (1.0 / math.sqrt(D))
    mi = mask.astype(jnp.int32)
    ik_ref[0, 0] = src + mi * N
    iq_ref[0, 0] = dst + mi * N + 2 * N
    iv_ref[0, 0] = src + 4 * N
    ip_ref[0, 0] = dst >> 3


def _edge_pre(edge_attr, src2, dst2, we, b2):
    nbe = E // BE
    idx_struct = jax.ShapeDtypeStruct((nbe, 1, BE), jnp.int32)
    idx_spec = pl.BlockSpec((1, 1, BE), lambda i: (i, 0, 0))
    return pl.pallas_call(
        _edge_body,
        grid=(nbe,),
        in_specs=[
            pl.BlockSpec((BE, IN_DIM), lambda i: (i, 0)),
            idx_spec,
            idx_spec,
            pl.BlockSpec((HD, IN_DIM), lambda i: (0, 0)),
            pl.BlockSpec((2, HD), lambda i: (0, 0)),
        ],
        out_specs=[
            pl.BlockSpec((BE, HD), lambda i: (i, 0)),
            idx_spec, idx_spec, idx_spec, idx_spec,
        ],
        out_shape=[
            jax.ShapeDtypeStruct((E, HD), jnp.float32),
            idx_struct, idx_struct, idx_struct, idx_struct,
        ],
    )(edge_attr, src2, dst2, we, b2)


@functools.lru_cache(maxsize=None)
def _make_sc_edge():
  mesh = plsc.VectorSubcoreMesh(core_axis_name="c", subcore_axis_name="s",
                                num_cores=NC, num_subcores=NS)

  @functools.partial(
      pl.kernel,
      out_type=(jax.ShapeDtypeStruct((NC, N2, HD), jnp.float32),
                jax.ShapeDtypeStruct((NC, NP2, HD), jnp.float32)),
      mesh=mesh,
      scratch_types=[
          pltpu.VMEM((CH,), jnp.int32),
          pltpu.VMEM((CH,), jnp.int32),
          pltpu.VMEM((CH,), jnp.int32),
          pltpu.VMEM((CH,), jnp.int32),
          pltpu.VMEM((CH + 8,), jnp.int32),
          pltpu.VMEM((CH,), jnp.int32),
          pltpu.VMEM((CH,), jnp.int32),
          pltpu.VMEM((CH,), jnp.int32),
          pltpu.VMEM((CH,), jnp.int32),
          pltpu.VMEM((CH,), jnp.int32),
          pltpu.VMEM((CH + 8,), jnp.int32),
          pltpu.VMEM((CH,), jnp.int32),
          pltpu.VMEM((CH, HD), jnp.float32),
          pltpu.VMEM((CH, HD), jnp.float32),
          pltpu.VMEM((CH, HD), jnp.float32),
          pltpu.VMEM((CH, HD), jnp.float32),
          pltpu.VMEM((CH, HD), jnp.float32),
          pltpu.VMEM((CH, HD), jnp.float32),
          pltpu.VMEM((2 * D,), jnp.float32),
          pltpu.VMEM((16,), jnp.int32),
          pltpu.VMEM((16, HD), jnp.float32),
          pltpu.VMEM_SHARED((N2, HD), jnp.float32),
          pltpu.VMEM_SHARED((NP2, HD), jnp.float32),
          pltpu.SemaphoreType.DMA,
          pltpu.SemaphoreType.DMA,
          pltpu.SemaphoreType.DMA,
          pltpu.SemaphoreType.DMA,
          pltpu.SemaphoreType.DMA,
      ],
  )
  def _sc_edge(proj, e_all, ik_all, iq_all, iv_all, dst_all, ip_all, rows_all,
               msg_out, ps_out,
               ikv, iqv, ivv, dstv, dstpad, ipv,
               ikv2, iqv2, ivv2, dstv2, dstpad2, ipv2,
               kbuf, qbuf, vbuf, ebuf,
               msgbuf, psbuf, tmp, idx16, wbuf, msgacc, psacc,
               semg, semi, semi2, semk, semq):
    c = lax.axis_index("c")
    s = lax.axis_index("s")
    wid = s * NC + c

    zv = jnp.zeros((D,), jnp.float32)
    lanes = lax.iota(jnp.int32, D)

    def _zw(r, carry):
        for j in range(H):
            wbuf[r, pl.ds(j * D, D)] = zv
        return carry
    lax.fori_loop(0, 16, _zw, 0)

    def _zc(b, carry):
        r0 = s * RPW + b * 16
        pltpu.sync_copy(rows_all.at[pl.ds(r0, 16)], idx16)
        pltpu.sync_copy(wbuf, msgacc.at[idx16])
        return carry
    lax.fori_loop(0, RPW // 16, _zc, 0)

    def _zp(b, carry):
        r0 = s * PPW + b * 16
        pltpu.sync_copy(rows_all.at[pl.ds(r0, 16)], idx16)
        pltpu.sync_copy(wbuf, psacc.at[idx16])
        return carry
    lax.fori_loop(0, PPW // 16, _zp, 0)

    tmp[pl.ds(D, D)] = zv

    plsc.subcore_barrier()

    base0 = wid * EPW
    setA = (ikv, iqv, ivv, dstv, dstpad, ipv)
    setB = (ikv2, iqv2, ivv2, dstv2, dstpad2, ipv2)

    def _fire_idx(base, bufs, sem):
        pltpu.async_copy(ik_all.at[pl.ds(base, CH)], bufs[0], sem)
        pltpu.async_copy(iq_all.at[pl.ds(base, CH)], bufs[1], sem)
        pltpu.async_copy(iv_all.at[pl.ds(base, CH)], bufs[2], sem)
        pltpu.async_copy(dst_all.at[pl.ds(base, CH)], bufs[3], sem)
        pltpu.async_copy(dst_all.at[pl.ds(base, CH)], bufs[4].at[pl.ds(0, CH)], sem)
        pltpu.async_copy(ip_all.at[pl.ds(base, CH)], bufs[5], sem)

    def _drain_idx(base, bufs, sem):
        pltpu.make_async_copy(ik_all.at[pl.ds(base, CH)], bufs[0], sem).wait()
        pltpu.make_async_copy(iq_all.at[pl.ds(base, CH)], bufs[1], sem).wait()
        pltpu.make_async_copy(iv_all.at[pl.ds(base, CH)], bufs[2], sem).wait()
        pltpu.make_async_copy(dst_all.at[pl.ds(base, CH)], bufs[3], sem).wait()
        pltpu.make_async_copy(dst_all.at[pl.ds(base, CH)], bufs[4].at[pl.ds(0, CH)], sem).wait()
        pltpu.make_async_copy(ip_all.at[pl.ds(base, CH)], bufs[5], sem).wait()

    _fire_idx(base0, setA, semi)

    HA, HB = 24, 16

    def _one_chunk(ci, bufs, nbufs, sem, nsem):
        base = base0 + ci * CH
        _drain_idx(base, bufs, sem)
        h0 = [
            pltpu.async_copy(proj.at[bufs[0].at[pl.ds(0, HA)]], kbuf.at[pl.ds(0, HA)], semk),
            pltpu.async_copy(proj.at[bufs[1].at[pl.ds(0, HA)]], qbuf.at[pl.ds(0, HA)], semk),
            pltpu.async_copy(proj.at[bufs[2].at[pl.ds(0, HA)]], vbuf.at[pl.ds(0, HA)], semk),
            pltpu.async_copy(e_all.at[pl.ds(base, CH)], ebuf, semg),
        ]
        h1 = [
            pltpu.async_copy(proj.at[bufs[0].at[pl.ds(HA, HB)]], kbuf.at[pl.ds(HA, HB)], semq),
            pltpu.async_copy(proj.at[bufs[1].at[pl.ds(HA, HB)]], qbuf.at[pl.ds(HA, HB)], semq),
            pltpu.async_copy(proj.at[bufs[2].at[pl.ds(HA, HB)]], vbuf.at[pl.ds(HA, HB)], semq),
        ]
        _fire_idx(base + CH, nbufs, nsem)
        for cpy in h0:
            cpy.wait()
        _compute(bufs, 0, HA)
        for cpy in h1:
            cpy.wait()
        _compute(bufs, HA, HB)
        pltpu.sync_copy(msgbuf, msgacc.at[bufs[3]], add=True)
        pltpu.sync_copy(psbuf, psacc.at[bufs[5]], add=True)

    def _compute(bufs, off, size):
        def _grp(g, gcarry):
            dvec = bufs[4][pl.ds(off + g * 8, D)]
            ovec = (dvec & 7) * D
            for l in range(8):
                ei = off + g * 8 + l
                acc = kbuf[ei, pl.ds(0, D)] * qbuf[ei, pl.ds(0, D)] * ebuf[ei, pl.ds(0, D)]
                for j in range(1, H):
                    sl = pl.ds(j * D, D)
                    acc = acc + kbuf[ei, sl] * qbuf[ei, sl] * ebuf[ei, sl]
                tmp[pl.ds(0, D)] = acc
                p = jnp.exp(acc + tmp[pl.ds(H, D)])
                for j in range(H):
                    psbuf[ei, pl.ds(j * D, D)] = zv
                pm = jnp.where(lanes < H, p, 0.0)
                psbuf[ei, pl.ds(ovec[l], D)] = pm
                for h in range(H):
                    pv = jnp.broadcast_to(p[h], (D,))
                    vv = vbuf[ei, pl.ds(h * D, D)]
                    msgbuf[ei, pl.ds(h * D, D)] = vv * pv
            return gcarry
        lax.fori_loop(0, size // 8, _grp, 0)

    def _pair(cj, carry):
        _one_chunk(2 * cj, setA, setB, semi, semi2)
        _one_chunk(2 * cj + 1, setB, setA, semi2, semi)
        return carry

    lax.fori_loop(0, NCHUNK // 2, _pair, 0)

    _drain_idx(base0 + NCHUNK * CH, setA, semi)

    plsc.subcore_barrier()

    def _wo(b, carry):
        r0 = s * RPW + b * 16
        pltpu.sync_copy(rows_all.at[pl.ds(r0, 16)], idx16)
        pltpu.sync_copy(msgacc.at[idx16], wbuf)
        pltpu.sync_copy(wbuf, msg_out.at[c, pl.ds(r0, 16)])
        return carry
    lax.fori_loop(0, RPW // 16, _wo, 0)

    def _wp(b, carry):
        r0 = s * PPW + b * 16
        pltpu.sync_copy(rows_all.at[pl.ds(r0, 16)], idx16)
        pltpu.sync_copy(psacc.at[idx16], wbuf)
        pltpu.sync_copy(wbuf, ps_out.at[c, pl.ds(r0, 16)])
        return carry
    lax.fori_loop(0, PPW // 16, _wp, 0)

  return _sc_edge


def _fin_body(msg_ref, ssum_ref, sel_ref, out_ref):
    msgs = msg_ref[0] + msg_ref[1]
    t = ssum_ref[0] + ssum_ref[1]
    denom = lax.dot_general(t[:, :H], sel_ref[...], (((1,), (0,)), ((), ())),
                            preferred_element_type=jnp.float32)
    out_ref[...] = msgs * 0.5 / (denom + 1e-16)


def _finalize(msgp, ssump, sel):
    return pl.pallas_call(
        _fin_body,
        grid=(N // BF,),
        in_specs=[
            pl.BlockSpec((NC, BF, HD), lambda i: (0, i, 0)),
            pl.BlockSpec((NC, BF, D), lambda i: (0, i, 0)),
            pl.BlockSpec((H, HD), lambda i: (0, 0)),
        ],
        out_specs=pl.BlockSpec((BF, HD), lambda i: (i, 0)),
        out_shape=jax.ShapeDtypeStruct((N, HD), jnp.float32),
    )(msgp, ssump, sel)


def kernel(x, edge_index, edge_attr, Wq, bq, Wk, bk, Wv, bv,
           Wqve, bqve, Wkve, bkve, We, be, Weve, beve):
    wstack = jnp.stack([Wk[_PERM], Wkve[_PERM], Wq[_PERM], Wqve[_PERM], Wv])
    bstack = jnp.stack([bk[_PERM], bkve[_PERM], bq[_PERM], bqve[_PERM], bv])
    proj = _projections(x, wstack, bstack).reshape(5 * N, HD)

    nbe = E // BE
    src2 = edge_index[0].reshape(nbe, 1, BE)
    dst2 = edge_index[1].reshape(nbe, 1, BE)
    b2 = jnp.stack([be[_PERM], beve[_PERM]])
    e_tot, ik3, iq3, iv3, ip3 = _edge_pre(edge_attr, src2, dst2, We[_PERM], b2)
    zpad = jnp.zeros((CH,), jnp.int32)
    ik = jnp.concatenate([ik3.reshape(E), zpad])
    iq = jnp.concatenate([iq3.reshape(E), zpad])
    iv = jnp.concatenate([iv3.reshape(E), zpad])
    ip = jnp.concatenate([ip3.reshape(E), zpad])
    dst = jnp.concatenate([edge_index[1], zpad])

    rows_all = jnp.arange(N2, dtype=jnp.int32)
    msgp, psp = _make_sc_edge()(proj, e_tot, ik, iq, iv, dst, ip, rows_all)
    ssump = psp.reshape(NC, N2, D)

    sel = jnp.repeat(jnp.eye(H, dtype=jnp.float32), D, axis=1)
    wv = _finalize(msgp, ssump, sel)
    return wv.reshape(N, H, D), edge_attr


# speedup vs baseline: 1.1605x; 1.1605x over previous
import functools
import math

import numpy as np
import jax
import jax.numpy as jnp
from jax import lax
from jax.experimental import pallas as pl
from jax.experimental.pallas import tpu as pltpu
from jax.experimental.pallas import tpu_sc as plsc

N = 10000
E = 320000
IN_DIM = 128
H = 8
D = 16
HD = H * D

NC = 2
NS = 16
NW = NC * NS
EPW = E // NW
CH = 40
NCHUNK = EPW // CH
N2 = 10240
RPW = N2 // NS
NP2 = N2 // 8
PPW = NP2 // NS

_ci = np.arange(HD)
_PERM = (((_ci % 16) % 8) * D + 2 * (_ci // 16) + (_ci % 16 >= 8)).astype(np.int32)

BN = 1000
BE = 2000
BF = 1000


def _proj_body(x_ref, w_ref, b_ref, out_ref):
    xb = x_ref[...]
    for j in range(5):
        acc = lax.dot_general(xb, w_ref[j], (((1,), (1,)), ((), ())),
                              preferred_element_type=jnp.float32)
        out_ref[j] = acc + b_ref[j][None, :]


def _projections(x, wstack, bstack):
    return pl.pallas_call(
        _proj_body,
        grid=(N // BN,),
        in_specs=[
            pl.BlockSpec((BN, IN_DIM), lambda i: (i, 0)),
            pl.BlockSpec((5, HD, IN_DIM), lambda i: (0, 0, 0)),
            pl.BlockSpec((5, HD), lambda i: (0, 0)),
        ],
        out_specs=pl.BlockSpec((5, BN, HD), lambda i: (0, i, 0)),
        out_shape=jax.ShapeDtypeStruct((5, N, HD), jnp.float32),
    )(x, wstack, bstack)


def _edge_body(attr_ref, src_ref, dst_ref, we_ref, b2_ref,
               e_ref, ik_ref, iq_ref, iv_ref, ip_ref):
    attr = attr_ref[...]
    src = src_ref[0, 0]
    dst = dst_ref[0, 0]
    p = lax.dot_general(attr, we_ref[...], (((1,), (1,)), ((), ())),
                        preferred_element_type=jnp.float32)
    p = p + b2_ref[0][None, :]
    mask = jnp.max(jnp.abs(attr), axis=1) == 0.0
    e_ref[...] = jnp.where(mask[:, None], b2_ref[1][None, :], p) * (1.0 / math.sqrt(D))
    mi = mask.astype(jnp.int32)
    ik_ref[0, 0] = src + mi * N
    iq_ref[0, 0] = dst + mi * N + 2 * N
    iv_ref[0, 0] = src + 4 * N
    ip_ref[0, 0] = dst >> 3


def _edge_pre(edge_attr, src2, dst2, we, b2):
    nbe = E // BE
    idx_struct = jax.ShapeDtypeStruct((nbe, 1, BE), jnp.int32)
    idx_spec = pl.BlockSpec((1, 1, BE), lambda i: (i, 0, 0))
    return pl.pallas_call(
        _edge_body,
        grid=(nbe,),
        in_specs=[
            pl.BlockSpec((BE, IN_DIM), lambda i: (i, 0)),
            idx_spec,
            idx_spec,
            pl.BlockSpec((HD, IN_DIM), lambda i: (0, 0)),
            pl.BlockSpec((2, HD), lambda i: (0, 0)),
        ],
        out_specs=[
            pl.BlockSpec((BE, HD), lambda i: (i, 0)),
            idx_spec, idx_spec, idx_spec, idx_spec,
        ],
        out_shape=[
            jax.ShapeDtypeStruct((E, HD), jnp.float32),
            idx_struct, idx_struct, idx_struct, idx_struct,
        ],
    )(edge_attr, src2, dst2, we, b2)


@functools.lru_cache(maxsize=None)
def _make_sc_edge():
  mesh = plsc.VectorSubcoreMesh(core_axis_name="c", subcore_axis_name="s",
                                num_cores=NC, num_subcores=NS)

  @functools.partial(
      pl.kernel,
      out_type=(jax.ShapeDtypeStruct((NC, N2, HD), jnp.float32),
                jax.ShapeDtypeStruct((NC, NP2, HD), jnp.float32)),
      mesh=mesh,
      scratch_types=[
          pltpu.VMEM((CH,), jnp.int32),
          pltpu.VMEM((CH,), jnp.int32),
          pltpu.VMEM((CH,), jnp.int32),
          pltpu.VMEM((CH,), jnp.int32),
          pltpu.VMEM((CH + 8,), jnp.int32),
          pltpu.VMEM((CH,), jnp.int32),
          pltpu.VMEM((CH,), jnp.int32),
          pltpu.VMEM((CH,), jnp.int32),
          pltpu.VMEM((CH,), jnp.int32),
          pltpu.VMEM((CH,), jnp.int32),
          pltpu.VMEM((CH + 8,), jnp.int32),
          pltpu.VMEM((CH,), jnp.int32),
          pltpu.VMEM((CH, HD), jnp.float32),
          pltpu.VMEM((CH, HD), jnp.float32),
          pltpu.VMEM((CH, HD), jnp.float32),
          pltpu.VMEM((CH, HD), jnp.float32),
          pltpu.VMEM((CH, HD), jnp.float32),
          pltpu.VMEM((CH, HD), jnp.float32),
          pltpu.VMEM((2 * D,), jnp.float32),
          pltpu.VMEM((16,), jnp.int32),
          pltpu.VMEM((16, HD), jnp.float32),
          pltpu.VMEM_SHARED((N2, HD), jnp.float32),
          pltpu.VMEM_SHARED((NP2, HD), jnp.float32),
          pltpu.SemaphoreType.DMA,
          pltpu.SemaphoreType.DMA,
          pltpu.SemaphoreType.DMA,
          pltpu.SemaphoreType.DMA,
          pltpu.SemaphoreType.DMA,
      ],
  )
  def _sc_edge(proj, e_all, ik_all, iq_all, iv_all, dst_all, ip_all, rows_all,
               msg_out, ps_out,
               ikv, iqv, ivv, dstv, dstpad, ipv,
               ikv2, iqv2, ivv2, dstv2, dstpad2, ipv2,
               kbuf, qbuf, vbuf, ebuf,
               msgbuf, psbuf, tmp, idx16, wbuf, msgacc, psacc,
               semg, semi, semi2, semk, semq):
    c = lax.axis_index("c")
    s = lax.axis_index("s")
    wid = s * NC + c

    zv = jnp.zeros((D,), jnp.float32)
    lanes = lax.iota(jnp.int32, D)

    def _zw(r, carry):
        for j in range(H):
            wbuf[r, pl.ds(j * D, D)] = zv
        return carry
    lax.fori_loop(0, 16, _zw, 0)

    def _zc(b, carry):
        r0 = s * RPW + b * 16
        pltpu.sync_copy(rows_all.at[pl.ds(r0, 16)], idx16)
        pltpu.sync_copy(wbuf, msgacc.at[idx16])
        return carry
    lax.fori_loop(0, RPW // 16, _zc, 0)

    def _zp(b, carry):
        r0 = s * PPW + b * 16
        pltpu.sync_copy(rows_all.at[pl.ds(r0, 16)], idx16)
        pltpu.sync_copy(wbuf, psacc.at[idx16])
        return carry
    lax.fori_loop(0, PPW // 16, _zp, 0)

    tmp[pl.ds(D, D)] = zv

    plsc.subcore_barrier()

    base0 = wid * EPW
    setA = (ikv, iqv, ivv, dstv, dstpad, ipv)
    setB = (ikv2, iqv2, ivv2, dstv2, dstpad2, ipv2)

    def _fire_idx(base, bufs, sem):
        pltpu.async_copy(ik_all.at[pl.ds(base, CH)], bufs[0], sem)
        pltpu.async_copy(iq_all.at[pl.ds(base, CH)], bufs[1], sem)
        pltpu.async_copy(iv_all.at[pl.ds(base, CH)], bufs[2], sem)
        pltpu.async_copy(dst_all.at[pl.ds(base, CH)], bufs[3], sem)
        pltpu.async_copy(dst_all.at[pl.ds(base, CH)], bufs[4].at[pl.ds(0, CH)], sem)
        pltpu.async_copy(ip_all.at[pl.ds(base, CH)], bufs[5], sem)

    def _drain_idx(base, bufs, sem):
        pltpu.make_async_copy(ik_all.at[pl.ds(base, CH)], bufs[0], sem).wait()
        pltpu.make_async_copy(iq_all.at[pl.ds(base, CH)], bufs[1], sem).wait()
        pltpu.make_async_copy(iv_all.at[pl.ds(base, CH)], bufs[2], sem).wait()
        pltpu.make_async_copy(dst_all.at[pl.ds(base, CH)], bufs[3], sem).wait()
        pltpu.make_async_copy(dst_all.at[pl.ds(base, CH)], bufs[4].at[pl.ds(0, CH)], sem).wait()
        pltpu.make_async_copy(ip_all.at[pl.ds(base, CH)], bufs[5], sem).wait()

    _fire_idx(base0, setA, semi)

    def _drain_scatter(bufs):
        pltpu.make_async_copy(msgbuf, msgacc.at[bufs[3]], semq).wait()
        pltpu.make_async_copy(psbuf, psacc.at[bufs[5]], semq).wait()

    def _one_chunk(ci, bufs, nbufs, sem, nsem, pbufs, guard):
        base = base0 + ci * CH
        _drain_idx(base, bufs, sem)
        gs = [
            pltpu.async_copy(proj.at[bufs[0]], kbuf, semk),
            pltpu.async_copy(proj.at[bufs[1]], qbuf, semk),
            pltpu.async_copy(proj.at[bufs[2]], vbuf, semk),
            pltpu.async_copy(e_all.at[pl.ds(base, CH)], ebuf, semg),
        ]
        if guard is None:
            _drain_scatter(pbufs)
        else:
            @pl.when(guard)
            def _():
                _drain_scatter(pbufs)
        _fire_idx(base + CH, nbufs, nsem)
        for cpy in gs:
            cpy.wait()
        _compute(bufs, 0, CH)
        pltpu.async_copy(msgbuf, msgacc.at[bufs[3]], semq, add=True)
        pltpu.async_copy(psbuf, psacc.at[bufs[5]], semq, add=True)

    def _compute(bufs, off, size):
        def _grp(g, gcarry):
            dvec = bufs[4][pl.ds(off + g * 8, D)]
            ovec = (dvec & 7) * D
            for l in range(8):
                ei = off + g * 8 + l
                acc = kbuf[ei, pl.ds(0, D)] * qbuf[ei, pl.ds(0, D)] * ebuf[ei, pl.ds(0, D)]
                for j in range(1, H):
                    sl = pl.ds(j * D, D)
                    acc = acc + kbuf[ei, sl] * qbuf[ei, sl] * ebuf[ei, sl]
                tmp[pl.ds(0, D)] = acc
                p = jnp.exp(acc + tmp[pl.ds(H, D)])
                for j in range(H):
                    psbuf[ei, pl.ds(j * D, D)] = zv
                pm = jnp.where(lanes < H, p, 0.0)
                psbuf[ei, pl.ds(ovec[l], D)] = pm
                for h in range(H):
                    pv = jnp.broadcast_to(p[h], (D,))
                    vv = vbuf[ei, pl.ds(h * D, D)]
                    msgbuf[ei, pl.ds(h * D, D)] = vv * pv
            return gcarry
        lax.fori_loop(0, size // 8, _grp, 0)

    def _pair(cj, carry):
        _one_chunk(2 * cj, setA, setB, semi, semi2, setB, cj > 0)
        _one_chunk(2 * cj + 1, setB, setA, semi2, semi, setA, None)
        return carry

    lax.fori_loop(0, NCHUNK // 2, _pair, 0)

    _drain_idx(base0 + NCHUNK * CH, setA, semi)
    _drain_scatter(setB)

    plsc.subcore_barrier()

    def _wo(b, carry):
        r0 = s * RPW + b * 16
        pltpu.sync_copy(rows_all.at[pl.ds(r0, 16)], idx16)
        pltpu.sync_copy(msgacc.at[idx16], wbuf)
        pltpu.sync_copy(wbuf, msg_out.at[c, pl.ds(r0, 16)])
        return carry
    lax.fori_loop(0, RPW // 16, _wo, 0)

    def _wp(b, carry):
        r0 = s * PPW + b * 16
        pltpu.sync_copy(rows_all.at[pl.ds(r0, 16)], idx16)
        pltpu.sync_copy(psacc.at[idx16], wbuf)
        pltpu.sync_copy(wbuf, ps_out.at[c, pl.ds(r0, 16)])
        return carry
    lax.fori_loop(0, PPW // 16, _wp, 0)

  return _sc_edge


def _fin_body(msg_ref, ssum_ref, sel_ref, out_ref):
    msgs = msg_ref[0] + msg_ref[1]
    t = ssum_ref[0] + ssum_ref[1]
    denom = lax.dot_general(t[:, :H], sel_ref[...], (((1,), (0,)), ((), ())),
                            preferred_element_type=jnp.float32)
    out_ref[...] = msgs * 0.5 / (denom + 1e-16)


def _finalize(msgp, ssump, sel):
    return pl.pallas_call(
        _fin_body,
        grid=(N // BF,),
        in_specs=[
            pl.BlockSpec((NC, BF, HD), lambda i: (0, i, 0)),
            pl.BlockSpec((NC, BF, D), lambda i: (0, i, 0)),
            pl.BlockSpec((H, HD), lambda i: (0, 0)),
        ],
        out_specs=pl.BlockSpec((BF, HD), lambda i: (i, 0)),
        out_shape=jax.ShapeDtypeStruct((N, HD), jnp.float32),
    )(msgp, ssump, sel)


def kernel(x, edge_index, edge_attr, Wq, bq, Wk, bk, Wv, bv,
           Wqve, bqve, Wkve, bkve, We, be, Weve, beve):
    wstack = jnp.stack([Wk[_PERM], Wkve[_PERM], Wq[_PERM], Wqve[_PERM], Wv])
    bstack = jnp.stack([bk[_PERM], bkve[_PERM], bq[_PERM], bqve[_PERM], bv])
    proj = _projections(x, wstack, bstack).reshape(5 * N, HD)

    nbe = E // BE
    src2 = edge_index[0].reshape(nbe, 1, BE)
    dst2 = edge_index[1].reshape(nbe, 1, BE)
    b2 = jnp.stack([be[_PERM], beve[_PERM]])
    e_tot, ik3, iq3, iv3, ip3 = _edge_pre(edge_attr, src2, dst2, We[_PERM], b2)
    zpad = jnp.zeros((CH,), jnp.int32)
    ik = jnp.concatenate([ik3.reshape(E), zpad])
    iq = jnp.concatenate([iq3.reshape(E), zpad])
    iv = jnp.concatenate([iv3.reshape(E), zpad])
    ip = jnp.concatenate([ip3.reshape(E), zpad])
    dst = jnp.concatenate([edge_index[1], zpad])

    rows_all = jnp.arange(N2, dtype=jnp.int32)
    msgp, psp = _make_sc_edge()(proj, e_tot, ik, iq, iv, dst, ip, rows_all)
    ssump = psp.reshape(NC, N2, D)

    sel = jnp.repeat(jnp.eye(H, dtype=jnp.float32), D, axis=1)
    wv = _finalize(msgp, ssump, sel)
    return wv.reshape(N, H, D), edge_attr
